# baseline (speedup 1.0000x reference)
import functools

import jax
import jax.numpy as jnp
from jax import lax
from jax.experimental import pallas as pl
from jax.experimental.pallas import tpu as pltpu
from jax.experimental.pallas import tpu_sc as plsc

_N = 10000
_E = 160000
_HID = 256
_HEADS = (4, 4, 1)
_OUTC = (256, 256, 2)
_IND = (256, 1024, 1024)

_NPB = 32
_RPAD = 320
_NPAD = _RPAD * _NPB
_CH = 32
_CPAIR = 2
_ACCR = 32
_EP = 180224


def _stats_body(x_ref, s_ref, q_ref):
  @pl.when(pl.program_id(0) == 0)
  def _():
    s_ref[...] = jnp.zeros_like(s_ref)
    q_ref[...] = jnp.zeros_like(q_ref)

  xb = x_ref[...]
  s_ref[...] += jnp.sum(xb, axis=0, keepdims=True)
  q_ref[...] += jnp.sum(xb * xb, axis=0, keepdims=True)


def _stats(x):
  ind = x.shape[1]
  s, q = pl.pallas_call(
      _stats_body,
      grid=(_NPAD // 256,),
      in_specs=[pl.BlockSpec((256, ind), lambda m: (m, 0))],
      out_specs=[pl.BlockSpec((1, ind), lambda m: (0, 0))] * 2,
      out_shape=[jax.ShapeDtypeStruct((1, ind), jnp.float32)] * 2,
  )(x)
  mu = s[0] / _N
  var = q[0] / _N - mu * mu
  return mu, var


def _mm_body(x_ref, wh_ref, bh_ref, ws_ref, bs_ref, wm_ref, bm_ref,
             h_ref, sk_ref, sb_ref):
  xb = x_ref[...]
  h_ref[...] = jnp.dot(xb, wh_ref[...], preferred_element_type=jnp.float32, precision=lax.Precision.HIGHEST) + bh_ref[...]
  sk_ref[...] = jnp.dot(xb, ws_ref[...], preferred_element_type=jnp.float32, precision=lax.Precision.HIGHEST) + bs_ref[...]
  sb_ref[...] = jnp.dot(xb, wm_ref[...], preferred_element_type=jnp.float32, precision=lax.Precision.HIGHEST) + bm_ref[...]


def _mm(x, wh, bh, ws, bs, wm, bm):
  ind = x.shape[1]
  wd = wh.shape[1]
  full = lambda m: (0, 0)
  row = lambda m: (m, 0)
  return pl.pallas_call(
      _mm_body,
      grid=(_NPAD // 256,),
      in_specs=[
          pl.BlockSpec((256, ind), row),
          pl.BlockSpec((ind, wd), full),
          pl.BlockSpec((1, wd), full),
          pl.BlockSpec((ind, wd), full),
          pl.BlockSpec((1, wd), full),
          pl.BlockSpec((ind, 128), full),
          pl.BlockSpec((1, 128), full),
      ],
      out_specs=[
          pl.BlockSpec((256, wd), row),
          pl.BlockSpec((256, wd), row),
          pl.BlockSpec((256, 128), row),
      ],
      out_shape=[
          jax.ShapeDtypeStruct((_NPAD, wd), jnp.float32),
          jax.ShapeDtypeStruct((_NPAD, wd), jnp.float32),
          jax.ShapeDtypeStruct((_NPAD, 128), jnp.float32),
      ],
  )(x, wh, bh, ws, bs, wm, bm)


def _eattn_body(ea_ref, tw_ref, tb_ref, a1_ref, a2_ref, bt_ref, o0, o1, o2):
  ea = ea_ref[...]
  temb = jnp.cos(ea[:, 0:1] * tw_ref[...] + tb_ref[...])
  r = jnp.dot(temb, a2_ref[...], preferred_element_type=jnp.float32, precision=lax.Precision.HIGHEST)
  r += jnp.dot(ea, a1_ref[...], preferred_element_type=jnp.float32, precision=lax.Precision.HIGHEST)
  r += bt_ref[...]
  o0[...] = r[:, 0:16]
  o1[...] = r[:, 16:32]
  o2[...] = r[:, 32:48]


def _eattn(edge_attr, te_w, te_b, a1, a2, bt):
  full = lambda m: (0, 0)
  row = lambda m: (m, 0)
  return pl.pallas_call(
      _eattn_body,
      grid=(_EP // 2048,),
      in_specs=[
          pl.BlockSpec((2048, 16), row),
          pl.BlockSpec((1, _HID), full),
          pl.BlockSpec((1, _HID), full),
          pl.BlockSpec((16, 48), full),
          pl.BlockSpec((_HID, 48), full),
          pl.BlockSpec((1, 48), full),
      ],
      out_specs=[pl.BlockSpec((2048, 16), row)] * 3,
      out_shape=[jax.ShapeDtypeStruct((_EP, 16), jnp.float32)] * 3,
  )(edge_attr, te_w.reshape(1, _HID), te_b.reshape(1, _HID), a1, a2, bt)


def _sc_body(H, C, wd,
             h_hbm, sk_hbm, sb_hbm, ea_hbm, src_hbm, dl_hbm,
             est_hbm, nch_hbm, out_hbm,
             sdst_b, sb_bufs, ea_b, exal_b, den_b, acc_b, h_bufs,
             src_bufs, dl_bufs, dlv_b, meta_b, semi, semg, semh):
  cid = lax.axis_index("c")
  sid = lax.axis_index("s")
  wid = cid * 16 + sid
  nsl = wd // 16
  zv = jnp.zeros((16,), jnp.float32)
  iota = lax.iota(jnp.int32, 16)

  def chunk_off(estart, ci):
    return pl.multiple_of(estart + ci * _CH, _CH)

  def issue_idx(estart, ci, half):
    eo = chunk_off(estart, ci)
    pltpu.async_copy(src_hbm.at[pl.ds(eo, _CH)], src_bufs[half], semi)
    pltpu.async_copy(dl_hbm.at[pl.ds(eo, _CH)], dl_bufs[half], semi)

  def wait_idx(half):
    pltpu.make_async_copy(src_hbm.at[pl.ds(0, _CH)], src_bufs[half], semi).wait()
    pltpu.make_async_copy(dl_hbm.at[pl.ds(0, _CH)], dl_bufs[half], semi).wait()

  def issue_gathers(half, with_h):
    pltpu.async_copy(sb_hbm.at[src_bufs[half]], sb_bufs[half], semg)
    if with_h:
      pltpu.async_copy(h_hbm.at[src_bufs[half]], h_bufs[half], semh)

  def wait_gather(half):
    pltpu.make_async_copy(sb_hbm.at[src_bufs[half]], sb_bufs[half], semg).wait()

  def wait_h(half):
    pltpu.make_async_copy(h_hbm.at[src_bufs[half]], h_bufs[half], semh).wait()

  def edge_chunk_logits(g, half):
    rowi = iota + g * 16
    dl = dl_bufs[half][pl.ds(g * 16, 16)]
    isdum = dl >= jnp.int32(_NPB)
    dlv = jnp.where(isdum, jnp.int32(0), dl)
    exs = []
    for j in range(H):
      jf = jnp.full((16,), j, jnp.int32)
      s1 = plsc.load_gather(sb_bufs[half], [rowi, jf])
      e1 = plsc.load_gather(ea_b, [rowi, jf])
      s2 = plsc.load_gather(sdst_b, [dlv, jf + 8])
      al = s1 + s2 + e1
      al = jnp.where(al < 0.0, al * jnp.float32(0.2), al)
      exs.append(jnp.where(isdum, jnp.float32(0.0), jnp.exp(al)))
    return rowi, dlv, exs

  def range_step(rr, carry):
    r = wid + rr * 32
    pltpu.sync_copy(est_hbm.at[r], meta_b)
    estart = meta_b[...][0]
    pltpu.sync_copy(nch_hbm.at[r], meta_b)
    nchunks = meta_b[...][0]
    nstart = pl.multiple_of(r * _NPB, _NPB)

    def zrow(i, _):
      for k in range(nsl):
        acc_b[i, pl.ds(k * 16, 16)] = zv
      den_b[i, pl.ds(0, 16)] = zv
      return 0
    lax.fori_loop(0, _NPB, zrow, 0)
    pltpu.sync_copy(sb_hbm.at[pl.ds(nstart, _NPB)], sdst_b)

    def chunk_compute(ci, half, is_b):
      eo = chunk_off(estart, ci)
      pltpu.sync_copy(ea_hbm.at[pl.ds(eo, _CH)], ea_b)
      wait_gather(half)
      for g in range(_CH // 16):
        rowi, dlv, exs = edge_chunk_logits(g, half)
        dlv_b[pl.ds(g * 16, 16)] = dlv
        for j in range(H):
          jf = jnp.full((16,), j, jnp.int32)
          if is_b:
            dv = plsc.load_gather(den_b, [dlv, jf])
            val = exs[j] / (dv + jnp.float32(1e-16))
          else:
            val = exs[j]
          plsc.store_scatter(exal_b, [rowi, jf], val)
      if is_b:
        wait_h(half)
        def wrow(e, _):
          ef = jnp.full((16,), e, jnp.int32)
          drep = plsc.load_gather(dlv_b, [ef])
          hb = h_bufs[half]
          if C >= 16:
            for j in range(H):
              av = plsc.load_gather(exal_b, [ef, jnp.full((16,), j, jnp.int32)])
              for k in range(C // 16):
                off = j * C + k * 16
                plsc.addupdate_scatter(
                    acc_b, [drep, iota + off], hb[e, pl.ds(off, 16)] * av)
          else:
            av = plsc.load_gather(exal_b, [ef, jnp.full((16,), 0, jnp.int32)])
            plsc.addupdate_scatter(acc_b, [drep, iota],
                                   hb[e, pl.ds(0, 16)] * av)
          return 0
        lax.fori_loop(0, _CH, wrow, 0)
      else:
        def arow(e, _):
          ef = jnp.full((16,), e, jnp.int32)
          drep = plsc.load_gather(dlv_b, [ef])
          plsc.addupdate_scatter(den_b, [drep, iota], exal_b[e, pl.ds(0, 16)])
          return 0
        lax.fori_loop(0, _CH, arow, 0)

    def run_pass(is_b):
      @pl.when(nchunks > 0)
      def _():
        issue_idx(estart, 0, 0)
        wait_idx(0)
        issue_gathers(0, is_b)
        def step(t, _):
          for half in range(_CPAIR):
            ci = t * _CPAIR + half
            nhalf = 1 - half
            @pl.when(ci + 1 < nchunks)
            def _():
              issue_idx(estart, ci + 1, nhalf)
              wait_idx(nhalf)
              issue_gathers(nhalf, is_b)
            chunk_compute(ci, half, is_b)
          return 0
        lax.fori_loop(0, nchunks // _CPAIR, step, 0)

    run_pass(False)
    run_pass(True)

    pltpu.sync_copy(sk_hbm.at[pl.ds(nstart, _NPB)], h_bufs[0])
    def orow(i, _):
      valid = jnp.where(nstart + i < _N, jnp.float32(1.0), jnp.float32(0.0))
      for k in range(nsl):
        sl = pl.ds(k * 16, 16)
        acc_b[i, sl] = jnp.maximum(acc_b[i, sl] + h_bufs[0][i, sl], 0.0) * valid
      return 0
    lax.fori_loop(0, _NPB, orow, 0)
    pltpu.sync_copy(acc_b, out_hbm.at[pl.ds(nstart, _NPB)])
    return carry

  lax.fori_loop(0, _RPAD // 32, range_step, 0)


def _sc_entry(H, C, wd,
              h_hbm, sk_hbm, sb_hbm, ea_hbm, src_hbm, dl_hbm,
              est_hbm, nch_hbm, out_hbm,
              sdst_b, sb0, sb1, ea_b, exal_b, den_b, acc_b, h0, h1,
              s0, s1, d0, d1, dlv_b, meta_b, semi, semg, semh):
  _sc_body(H, C, wd, h_hbm, sk_hbm, sb_hbm, ea_hbm, src_hbm, dl_hbm,
           est_hbm, nch_hbm, out_hbm,
           sdst_b, (sb0, sb1), ea_b, exal_b, den_b, acc_b, (h0, h1),
           (s0, s1), (d0, d1), dlv_b, meta_b, semi, semg, semh)


def _make_sc(H, C, wd):
  mesh = plsc.VectorSubcoreMesh(core_axis_name="c", subcore_axis_name="s")
  body = functools.partial(_sc_entry, H, C, wd)
  return pl.kernel(
      body,
      out_type=jax.ShapeDtypeStruct((_NPAD, wd), jnp.float32),
      mesh=mesh,
      compiler_params=pltpu.CompilerParams(needs_layout_passes=False),
      scratch_types=[
          pltpu.VMEM((_NPB, 128), jnp.float32),
          pltpu.VMEM((_CH, 128), jnp.float32),
          pltpu.VMEM((_CH, 128), jnp.float32),
          pltpu.VMEM((_CH, 16), jnp.float32),
          pltpu.VMEM((_CH, 16), jnp.float32),
          pltpu.VMEM((_NPB, 16), jnp.float32),
          pltpu.VMEM((_NPB, wd), jnp.float32),
          pltpu.VMEM((_CH, wd), jnp.float32),
          pltpu.VMEM((_CH, wd), jnp.float32),
          pltpu.VMEM((_CH,), jnp.int32),
          pltpu.VMEM((_CH,), jnp.int32),
          pltpu.VMEM((_CH,), jnp.int32),
          pltpu.VMEM((_CH,), jnp.int32),
          pltpu.VMEM((_CH,), jnp.int32),
          pltpu.VMEM((16,), jnp.int32),
          pltpu.SemaphoreType.DMA,
          pltpu.SemaphoreType.DMA,
          pltpu.SemaphoreType.DMA,
      ],
  )


_SC_KERNELS = {}


def _sc_call(H, C, wd, *args):
  key = (H, C, wd)
  if key not in _SC_KERNELS:
    _SC_KERNELS[key] = _make_sc(H, C, wd)
  return _SC_KERNELS[key](*args)


def kernel(x, edge_index, edge_attr, te_w, te_b, emlp_W, emlp_b,
           W0, Wedge0, asrc0, adst0, aedge0, b0, sW0, sb0, g0, be0,
           W1, Wedge1, asrc1, adst1, aedge1, b1, sW1, sb1, g1, be1,
           W2, Wedge2, asrc2, adst2, aedge2, b2, sW2, sb2, g2, be2):
  layers = [
      (W0, Wedge0, asrc0, adst0, aedge0, b0, sW0, sb0, g0, be0),
      (W1, Wedge1, asrc1, adst1, aedge1, b1, sW1, sb1, g1, be1),
      (W2, Wedge2, asrc2, adst2, aedge2, b2, sW2, sb2, g2, be2),
  ]

  src = edge_index[0]
  dst = edge_index[1]

  perm = jnp.argsort(dst)
  dst_s = dst[perm]
  src_s = src[perm]
  rid = dst_s // _NPB
  dstloc = dst_s - rid * _NPB
  counts = jnp.zeros((_RPAD,), jnp.int32).at[rid].add(1)
  cntp = ((counts + 2 * _CH - 1) // (2 * _CH)) * (2 * _CH)
  starts = jnp.concatenate([jnp.zeros((1,), jnp.int32),
                            jnp.cumsum(cntp)[:-1].astype(jnp.int32)])
  fidx = jnp.cumsum(counts).astype(jnp.int32) - counts
  pos = starts[rid] + (jnp.arange(_E, dtype=jnp.int32) - fidx[rid])
  src_pad = jnp.zeros((_EP,), jnp.int32).at[pos].set(src_s)
  dl_pad = jnp.full((_EP,), _NPB, jnp.int32).at[pos].set(dstloc)
  ea_pad = jnp.zeros((_EP, 16), jnp.float32).at[pos].set(edge_attr[perm])
  est_arr = jnp.broadcast_to(starts[:, None], (_RPAD, 16)).astype(jnp.int32)
  nch_arr = jnp.broadcast_to((cntp // _CH)[:, None], (_RPAD, 16)).astype(jnp.int32)

  a1s, a2s, bts = [], [], []
  for i in range(3):
    H, C = _HEADS[i], _OUTC[i]
    We = layers[i][1].reshape(_HID, H, C)
    Mi = jnp.einsum("khc,hc->kh", We, layers[i][4])
    a1 = jnp.zeros((16, 16), jnp.float32).at[1:16, :H].set((emlp_W[:15] @ Mi))
    a2 = jnp.zeros((_HID, 16), jnp.float32).at[:, :H].set(emlp_W[15:] @ Mi)
    bt = jnp.zeros((16,), jnp.float32).at[:H].set(emlp_b @ Mi)
    a1s.append(a1); a2s.append(a2); bts.append(bt)
  a1cat = jnp.concatenate(a1s, axis=1)
  a2cat = jnp.concatenate(a2s, axis=1)
  btcat = jnp.concatenate(bts).reshape(1, 48)
  ea0, ea1, ea2 = _eattn(ea_pad, te_w, te_b, a1cat, a2cat, btcat)
  eattns = [ea0, ea1, ea2]

  xp = jnp.pad(x, ((0, _NPAD - _N), (0, 0)))
  for i in range(3):
    W, _, asrc, adst, _, b, sW, sb, g, be = layers[i]
    H, C, ind = _HEADS[i], _OUTC[i], _IND[i]
    wd = max(H * C, 128)

    mu, var = _stats(xp)
    s = g * lax.rsqrt(var + 1e-5)
    shift = be - mu * s
    wh = s[:, None] * W
    bh = shift @ W
    ws = s[:, None] * sW
    bs = shift @ sW + sb + b
    if H * C < wd:
      wh = jnp.pad(wh, ((0, 0), (0, wd - H * C)))
      bh = jnp.pad(bh, (0, wd - H * C))
      ws = jnp.pad(ws, ((0, 0), (0, wd - H * C)))
      bs = jnp.pad(bs, (0, wd - H * C))
    masrc = jnp.einsum("khc,hc->kh", wh[:, :H * C].reshape(ind, H, C), asrc)
    madst = jnp.einsum("khc,hc->kh", wh[:, :H * C].reshape(ind, H, C), adst)
    bsrc = jnp.einsum("hc,hc->h", bh[:H * C].reshape(H, C), asrc)
    bdst = jnp.einsum("hc,hc->h", bh[:H * C].reshape(H, C), adst)
    wm = jnp.zeros((ind, 128), jnp.float32).at[:, :H].set(masrc).at[:, 8:8 + H].set(madst)
    bm = jnp.zeros((128,), jnp.float32).at[:H].set(bsrc).at[8:8 + H].set(bdst)

    h, skipb, sboth = _mm(xp, wh, bh.reshape(1, wd), ws, bs.reshape(1, wd),
                          wm, bm.reshape(1, 128))
    xp = _sc_call(H, C, wd, h, skipb, sboth, eattns[i],
                  src_pad, dl_pad, est_arr, nch_arr)

  return xp[:_N, :2]


# speedup vs baseline: 1.0159x; 1.0159x over previous
import functools

import jax
import jax.numpy as jnp
from jax import lax
from jax.experimental import pallas as pl
from jax.experimental.pallas import tpu as pltpu
from jax.experimental.pallas import tpu_sc as plsc

_N = 10000
_E = 160000
_HID = 256
_HEADS = (4, 4, 1)
_OUTC = (256, 256, 2)
_IND = (256, 1024, 1024)

_NPB = 32
_RPAD = 320
_NPAD = _RPAD * _NPB
_CH = 32
_CPAIR = 2
_ACCR = 32
_EP = 180224


def _stats_body(x_ref, s_ref, q_ref):
  @pl.when(pl.program_id(0) == 0)
  def _():
    s_ref[...] = jnp.zeros_like(s_ref)
    q_ref[...] = jnp.zeros_like(q_ref)

  xb = x_ref[...]
  s_ref[...] += jnp.sum(xb, axis=0, keepdims=True)
  q_ref[...] += jnp.sum(xb * xb, axis=0, keepdims=True)


def _stats(x):
  ind = x.shape[1]
  s, q = pl.pallas_call(
      _stats_body,
      grid=(_NPAD // 256,),
      in_specs=[pl.BlockSpec((256, ind), lambda m: (m, 0))],
      out_specs=[pl.BlockSpec((1, ind), lambda m: (0, 0))] * 2,
      out_shape=[jax.ShapeDtypeStruct((1, ind), jnp.float32)] * 2,
  )(x)
  mu = s[0] / _N
  var = q[0] / _N - mu * mu
  return mu, var


def _mm_body(x_ref, wh_ref, bh_ref, ws_ref, bs_ref, wm_ref, bm_ref,
             h_ref, sk_ref, sb_ref):
  xb = x_ref[...]
  h_ref[...] = jnp.dot(xb, wh_ref[...], preferred_element_type=jnp.float32, precision=lax.Precision.HIGHEST) + bh_ref[...]
  sk_ref[...] = jnp.dot(xb, ws_ref[...], preferred_element_type=jnp.float32, precision=lax.Precision.HIGHEST) + bs_ref[...]
  sb_ref[...] = jnp.dot(xb, wm_ref[...], preferred_element_type=jnp.float32, precision=lax.Precision.HIGHEST) + bm_ref[...]


def _mm(x, wh, bh, ws, bs, wm, bm):
  ind = x.shape[1]
  wd = wh.shape[1]
  full = lambda m: (0, 0)
  row = lambda m: (m, 0)
  return pl.pallas_call(
      _mm_body,
      grid=(_NPAD // 256,),
      in_specs=[
          pl.BlockSpec((256, ind), row),
          pl.BlockSpec((ind, wd), full),
          pl.BlockSpec((1, wd), full),
          pl.BlockSpec((ind, wd), full),
          pl.BlockSpec((1, wd), full),
          pl.BlockSpec((ind, 128), full),
          pl.BlockSpec((1, 128), full),
      ],
      out_specs=[
          pl.BlockSpec((256, wd), row),
          pl.BlockSpec((256, wd), row),
          pl.BlockSpec((256, 128), row),
      ],
      out_shape=[
          jax.ShapeDtypeStruct((_NPAD, wd), jnp.float32),
          jax.ShapeDtypeStruct((_NPAD, wd), jnp.float32),
          jax.ShapeDtypeStruct((_NPAD, 128), jnp.float32),
      ],
  )(x, wh, bh, ws, bs, wm, bm)


def _eattn_body(ea_ref, tw_ref, tb_ref, a1_ref, a2_ref, bt_ref, o0, o1, o2):
  ea = ea_ref[...]
  temb = jnp.cos(ea[:, 0:1] * tw_ref[...] + tb_ref[...])
  r = jnp.dot(temb, a2_ref[...], preferred_element_type=jnp.float32, precision=lax.Precision.HIGHEST)
  r += jnp.dot(ea, a1_ref[...], preferred_element_type=jnp.float32, precision=lax.Precision.HIGHEST)
  r += bt_ref[...]
  o0[...] = r[:, 0:16]
  o1[...] = r[:, 16:32]
  o2[...] = r[:, 32:48]


def _eattn(edge_attr, te_w, te_b, a1, a2, bt):
  full = lambda m: (0, 0)
  row = lambda m: (m, 0)
  return pl.pallas_call(
      _eattn_body,
      grid=(_EP // 2048,),
      in_specs=[
          pl.BlockSpec((2048, 16), row),
          pl.BlockSpec((1, _HID), full),
          pl.BlockSpec((1, _HID), full),
          pl.BlockSpec((16, 48), full),
          pl.BlockSpec((_HID, 48), full),
          pl.BlockSpec((1, 48), full),
      ],
      out_specs=[pl.BlockSpec((2048, 16), row)] * 3,
      out_shape=[jax.ShapeDtypeStruct((_EP, 16), jnp.float32)] * 3,
  )(edge_attr, te_w.reshape(1, _HID), te_b.reshape(1, _HID), a1, a2, bt)


def _sc_body(H, C, wd,
             h_hbm, sk_hbm, sb_hbm, ea_hbm, src_hbm, dl_hbm,
             est_hbm, nch_hbm, out_hbm,
             sdst_b, sb_bufs, ea_b, exal_b, den_b, acc_b, h_bufs,
             src_bufs, dl_bufs, dlv_b, meta_b, semi, semg, semh):
  cid = lax.axis_index("c")
  sid = lax.axis_index("s")
  wid = cid * 16 + sid
  nsl = wd // 16
  zv = jnp.zeros((16,), jnp.float32)
  iota = lax.iota(jnp.int32, 16)

  def chunk_off(estart, ci):
    return pl.multiple_of(estart + ci * _CH, _CH)

  def issue_idx(estart, ci, half):
    eo = chunk_off(estart, ci)
    pltpu.async_copy(src_hbm.at[pl.ds(eo, _CH)], src_bufs[half], semi)
    pltpu.async_copy(dl_hbm.at[pl.ds(eo, _CH)], dl_bufs[half], semi)

  def wait_idx(half):
    pltpu.make_async_copy(src_hbm.at[pl.ds(0, _CH)], src_bufs[half], semi).wait()
    pltpu.make_async_copy(dl_hbm.at[pl.ds(0, _CH)], dl_bufs[half], semi).wait()

  def issue_gathers(half, with_h):
    pltpu.async_copy(sb_hbm.at[src_bufs[half]], sb_bufs[half], semg)
    if with_h:
      pltpu.async_copy(h_hbm.at[src_bufs[half]], h_bufs[half], semh)

  def wait_gather(half):
    pltpu.make_async_copy(sb_hbm.at[src_bufs[half]], sb_bufs[half], semg).wait()

  def wait_h(half):
    pltpu.make_async_copy(h_hbm.at[src_bufs[half]], h_bufs[half], semh).wait()

  def edge_chunk_logits(g, half):
    rowi = iota + g * 16
    dl = dl_bufs[half][pl.ds(g * 16, 16)]
    isdum = dl >= jnp.int32(_NPB)
    dlv = jnp.where(isdum, jnp.int32(0), dl)
    exs = []
    for j in range(H):
      jf = jnp.full((16,), j, jnp.int32)
      s1 = plsc.load_gather(sb_bufs[half], [rowi, jf])
      e1 = plsc.load_gather(ea_b, [rowi, jf])
      s2 = plsc.load_gather(sdst_b, [dlv, jf + 8])
      al = s1 + s2 + e1
      al = jnp.where(al < 0.0, al * jnp.float32(0.2), al)
      exs.append(jnp.where(isdum, jnp.float32(0.0), jnp.exp(al)))
    return rowi, dlv, exs

  def range_step(rr, carry):
    r = wid + rr * 32
    pltpu.sync_copy(est_hbm.at[r], meta_b)
    estart = meta_b[...][0]
    pltpu.sync_copy(nch_hbm.at[r], meta_b)
    nchunks = meta_b[...][0]
    nstart = pl.multiple_of(r * _NPB, _NPB)

    def zrow(i, _):
      for k in range(nsl):
        acc_b[i, pl.ds(k * 16, 16)] = zv
      den_b[i, pl.ds(0, 16)] = zv
      return 0
    lax.fori_loop(0, _NPB, zrow, 0)
    pltpu.sync_copy(sb_hbm.at[pl.ds(nstart, _NPB)], sdst_b)

    def chunk_compute(ci, half, is_b, nchunks):
      eo = chunk_off(estart, ci)
      pltpu.sync_copy(ea_hbm.at[pl.ds(eo, _CH)], ea_b)
      wait_gather(half)
      for g in range(_CH // 16):
        rowi, dlv, exs = edge_chunk_logits(g, half)
        dlv_b[pl.ds(g * 16, 16)] = dlv
        for j in range(H):
          jf = jnp.full((16,), j, jnp.int32)
          if is_b:
            dv = plsc.load_gather(den_b, [dlv, jf])
            val = exs[j] / (dv + jnp.float32(1e-16))
          else:
            val = exs[j]
          plsc.store_scatter(exal_b, [rowi, jf], val)
      if is_b:
        wait_h(half)
      @pl.when(ci + 2 < nchunks)
      def _():
        issue_idx(estart, ci + 2, half)
      if is_b:
        def wrow(e, _):
          ef = jnp.full((16,), e, jnp.int32)
          drep = plsc.load_gather(dlv_b, [ef])
          hb = h_bufs[half]
          if C >= 16:
            for j in range(H):
              av = plsc.load_gather(exal_b, [ef, jnp.full((16,), j, jnp.int32)])
              for k in range(C // 16):
                off = j * C + k * 16
                plsc.addupdate_scatter(
                    acc_b, [drep, iota + off], hb[e, pl.ds(off, 16)] * av)
          else:
            av = plsc.load_gather(exal_b, [ef, jnp.full((16,), 0, jnp.int32)])
            plsc.addupdate_scatter(acc_b, [drep, iota],
                                   hb[e, pl.ds(0, 16)] * av)
          return 0
        lax.fori_loop(0, _CH, wrow, 0)
      else:
        def arow(e, _):
          ef = jnp.full((16,), e, jnp.int32)
          drep = plsc.load_gather(dlv_b, [ef])
          plsc.addupdate_scatter(den_b, [drep, iota], exal_b[e, pl.ds(0, 16)])
          return 0
        lax.fori_loop(0, _CH, arow, 0)

    def run_pass(is_b):
      @pl.when(nchunks > 0)
      def _():
        issue_idx(estart, 0, 0)
        wait_idx(0)
        issue_gathers(0, is_b)
        @pl.when(nchunks > 1)
        def _():
          issue_idx(estart, 1, 1)
        def step(t, _):
          for half in range(_CPAIR):
            ci = t * _CPAIR + half
            nhalf = 1 - half
            @pl.when(ci + 1 < nchunks)
            def _():
              wait_idx(nhalf)
              issue_gathers(nhalf, is_b)
            chunk_compute(ci, half, is_b, nchunks)
          return 0
        lax.fori_loop(0, nchunks // _CPAIR, step, 0)

    run_pass(False)
    run_pass(True)

    pltpu.sync_copy(sk_hbm.at[pl.ds(nstart, _NPB)], h_bufs[0])
    def orow(i, _):
      valid = jnp.where(nstart + i < _N, jnp.float32(1.0), jnp.float32(0.0))
      for k in range(nsl):
        sl = pl.ds(k * 16, 16)
        acc_b[i, sl] = jnp.maximum(acc_b[i, sl] + h_bufs[0][i, sl], 0.0) * valid
      return 0
    lax.fori_loop(0, _NPB, orow, 0)
    pltpu.sync_copy(acc_b, out_hbm.at[pl.ds(nstart, _NPB)])
    return carry

  lax.fori_loop(0, _RPAD // 32, range_step, 0)


def _sc_entry(H, C, wd,
              h_hbm, sk_hbm, sb_hbm, ea_hbm, src_hbm, dl_hbm,
              est_hbm, nch_hbm, out_hbm,
              sdst_b, sb0, sb1, ea_b, exal_b, den_b, acc_b, h0, h1,
              s0, s1, d0, d1, dlv_b, meta_b, semi, semg, semh):
  _sc_body(H, C, wd, h_hbm, sk_hbm, sb_hbm, ea_hbm, src_hbm, dl_hbm,
           est_hbm, nch_hbm, out_hbm,
           sdst_b, (sb0, sb1), ea_b, exal_b, den_b, acc_b, (h0, h1),
           (s0, s1), (d0, d1), dlv_b, meta_b, semi, semg, semh)


def _make_sc(H, C, wd):
  mesh = plsc.VectorSubcoreMesh(core_axis_name="c", subcore_axis_name="s")
  body = functools.partial(_sc_entry, H, C, wd)
  return pl.kernel(
      body,
      out_type=jax.ShapeDtypeStruct((_NPAD, wd), jnp.float32),
      mesh=mesh,
      compiler_params=pltpu.CompilerParams(needs_layout_passes=False),
      scratch_types=[
          pltpu.VMEM((_NPB, 128), jnp.float32),
          pltpu.VMEM((_CH, 128), jnp.float32),
          pltpu.VMEM((_CH, 128), jnp.float32),
          pltpu.VMEM((_CH, 16), jnp.float32),
          pltpu.VMEM((_CH, 16), jnp.float32),
          pltpu.VMEM((_NPB, 16), jnp.float32),
          pltpu.VMEM((_NPB, wd), jnp.float32),
          pltpu.VMEM((_CH, wd), jnp.float32),
          pltpu.VMEM((_CH, wd), jnp.float32),
          pltpu.VMEM((_CH,), jnp.int32),
          pltpu.VMEM((_CH,), jnp.int32),
          pltpu.VMEM((_CH,), jnp.int32),
          pltpu.VMEM((_CH,), jnp.int32),
          pltpu.VMEM((_CH,), jnp.int32),
          pltpu.VMEM((16,), jnp.int32),
          pltpu.SemaphoreType.DMA,
          pltpu.SemaphoreType.DMA,
          pltpu.SemaphoreType.DMA,
      ],
  )


_SC_KERNELS = {}


def _sc_call(H, C, wd, *args):
  key = (H, C, wd)
  if key not in _SC_KERNELS:
    _SC_KERNELS[key] = _make_sc(H, C, wd)
  return _SC_KERNELS[key](*args)


def kernel(x, edge_index, edge_attr, te_w, te_b, emlp_W, emlp_b,
           W0, Wedge0, asrc0, adst0, aedge0, b0, sW0, sb0, g0, be0,
           W1, Wedge1, asrc1, adst1, aedge1, b1, sW1, sb1, g1, be1,
           W2, Wedge2, asrc2, adst2, aedge2, b2, sW2, sb2, g2, be2):
  layers = [
      (W0, Wedge0, asrc0, adst0, aedge0, b0, sW0, sb0, g0, be0),
      (W1, Wedge1, asrc1, adst1, aedge1, b1, sW1, sb1, g1, be1),
      (W2, Wedge2, asrc2, adst2, aedge2, b2, sW2, sb2, g2, be2),
  ]

  src = edge_index[0]
  dst = edge_index[1]

  perm = jnp.argsort(dst)
  dst_s = dst[perm]
  src_s = src[perm]
  rid = dst_s // _NPB
  dstloc = dst_s - rid * _NPB
  counts = jnp.zeros((_RPAD,), jnp.int32).at[rid].add(1)
  cntp = ((counts + 2 * _CH - 1) // (2 * _CH)) * (2 * _CH)
  starts = jnp.concatenate([jnp.zeros((1,), jnp.int32),
                            jnp.cumsum(cntp)[:-1].astype(jnp.int32)])
  fidx = jnp.cumsum(counts).astype(jnp.int32) - counts
  pos = starts[rid] + (jnp.arange(_E, dtype=jnp.int32) - fidx[rid])
  src_pad = jnp.zeros((_EP,), jnp.int32).at[pos].set(src_s)
  dl_pad = jnp.full((_EP,), _NPB, jnp.int32).at[pos].set(dstloc)
  ea_pad = jnp.zeros((_EP, 16), jnp.float32).at[pos].set(edge_attr[perm])
  est_arr = jnp.broadcast_to(starts[:, None], (_RPAD, 16)).astype(jnp.int32)
  nch_arr = jnp.broadcast_to((cntp // _CH)[:, None], (_RPAD, 16)).astype(jnp.int32)

  a1s, a2s, bts = [], [], []
  for i in range(3):
    H, C = _HEADS[i], _OUTC[i]
    We = layers[i][1].reshape(_HID, H, C)
    Mi = jnp.einsum("khc,hc->kh", We, layers[i][4])
    a1 = jnp.zeros((16, 16), jnp.float32).at[1:16, :H].set((emlp_W[:15] @ Mi))
    a2 = jnp.zeros((_HID, 16), jnp.float32).at[:, :H].set(emlp_W[15:] @ Mi)
    bt = jnp.zeros((16,), jnp.float32).at[:H].set(emlp_b @ Mi)
    a1s.append(a1); a2s.append(a2); bts.append(bt)
  a1cat = jnp.concatenate(a1s, axis=1)
  a2cat = jnp.concatenate(a2s, axis=1)
  btcat = jnp.concatenate(bts).reshape(1, 48)
  ea0, ea1, ea2 = _eattn(ea_pad, te_w, te_b, a1cat, a2cat, btcat)
  eattns = [ea0, ea1, ea2]

  xp = jnp.pad(x, ((0, _NPAD - _N), (0, 0)))
  for i in range(3):
    W, _, asrc, adst, _, b, sW, sb, g, be = layers[i]
    H, C, ind = _HEADS[i], _OUTC[i], _IND[i]
    wd = max(H * C, 128)

    mu, var = _stats(xp)
    s = g * lax.rsqrt(var + 1e-5)
    shift = be - mu * s
    wh = s[:, None] * W
    bh = shift @ W
    ws = s[:, None] * sW
    bs = shift @ sW + sb + b
    if H * C < wd:
      wh = jnp.pad(wh, ((0, 0), (0, wd - H * C)))
      bh = jnp.pad(bh, (0, wd - H * C))
      ws = jnp.pad(ws, ((0, 0), (0, wd - H * C)))
      bs = jnp.pad(bs, (0, wd - H * C))
    masrc = jnp.einsum("khc,hc->kh", wh[:, :H * C].reshape(ind, H, C), asrc)
    madst = jnp.einsum("khc,hc->kh", wh[:, :H * C].reshape(ind, H, C), adst)
    bsrc = jnp.einsum("hc,hc->h", bh[:H * C].reshape(H, C), asrc)
    bdst = jnp.einsum("hc,hc->h", bh[:H * C].reshape(H, C), adst)
    wm = jnp.zeros((ind, 128), jnp.float32).at[:, :H].set(masrc).at[:, 8:8 + H].set(madst)
    bm = jnp.zeros((128,), jnp.float32).at[:H].set(bsrc).at[8:8 + H].set(bdst)

    h, skipb, sboth = _mm(xp, wh, bh.reshape(1, wd), ws, bs.reshape(1, wd),
                          wm, bm.reshape(1, 128))
    xp = _sc_call(H, C, wd, h, skipb, sboth, eattns[i],
                  src_pad, dl_pad, est_arr, nch_arr)

  return xp[:_N, :2]


# speedup vs baseline: 1.0463x; 1.0298x over previous
import functools

import jax
import jax.numpy as jnp
from jax import lax
from jax.experimental import pallas as pl
from jax.experimental.pallas import tpu as pltpu
from jax.experimental.pallas import tpu_sc as plsc

_N = 10000
_E = 160000
_HID = 256
_HEADS = (4, 4, 1)
_OUTC = (256, 256, 2)
_IND = (256, 1024, 1024)

_NPB = 32
_RPAD = 320
_NPAD = _RPAD * _NPB
_CH = 48
_ACCR = 32
_EP = 176128


def _stats_body(x_ref, s_ref, q_ref):
  @pl.when(pl.program_id(0) == 0)
  def _():
    s_ref[...] = jnp.zeros_like(s_ref)
    q_ref[...] = jnp.zeros_like(q_ref)

  xb = x_ref[...]
  s_ref[...] += jnp.sum(xb, axis=0, keepdims=True)
  q_ref[...] += jnp.sum(xb * xb, axis=0, keepdims=True)


def _stats(x):
  ind = x.shape[1]
  s, q = pl.pallas_call(
      _stats_body,
      grid=(_NPAD // 256,),
      in_specs=[pl.BlockSpec((256, ind), lambda m: (m, 0))],
      out_specs=[pl.BlockSpec((1, ind), lambda m: (0, 0))] * 2,
      out_shape=[jax.ShapeDtypeStruct((1, ind), jnp.float32)] * 2,
  )(x)
  mu = s[0] / _N
  var = q[0] / _N - mu * mu
  return mu, var


def _mm_body(x_ref, wh_ref, bh_ref, ws_ref, bs_ref, wm_ref, bm_ref,
             h_ref, sk_ref, sb_ref):
  xb = x_ref[...]
  h_ref[...] = jnp.dot(xb, wh_ref[...], preferred_element_type=jnp.float32, precision=lax.Precision.HIGHEST) + bh_ref[...]
  sk_ref[...] = jnp.dot(xb, ws_ref[...], preferred_element_type=jnp.float32, precision=lax.Precision.HIGHEST) + bs_ref[...]
  sb_ref[...] = jnp.dot(xb, wm_ref[...], preferred_element_type=jnp.float32, precision=lax.Precision.HIGHEST) + bm_ref[...]


def _mm(x, wh, bh, ws, bs, wm, bm):
  ind = x.shape[1]
  wd = wh.shape[1]
  full = lambda m: (0, 0)
  row = lambda m: (m, 0)
  return pl.pallas_call(
      _mm_body,
      grid=(_NPAD // 256,),
      in_specs=[
          pl.BlockSpec((256, ind), row),
          pl.BlockSpec((ind, wd), full),
          pl.BlockSpec((1, wd), full),
          pl.BlockSpec((ind, wd), full),
          pl.BlockSpec((1, wd), full),
          pl.BlockSpec((ind, 128), full),
          pl.BlockSpec((1, 128), full),
      ],
      out_specs=[
          pl.BlockSpec((256, wd), row),
          pl.BlockSpec((256, wd), row),
          pl.BlockSpec((256, 128), row),
      ],
      out_shape=[
          jax.ShapeDtypeStruct((_NPAD, wd), jnp.float32),
          jax.ShapeDtypeStruct((_NPAD, wd), jnp.float32),
          jax.ShapeDtypeStruct((_NPAD, 128), jnp.float32),
      ],
  )(x, wh, bh, ws, bs, wm, bm)


def _eattn_body(ea_ref, tw_ref, tb_ref, a1_ref, a2_ref, bt_ref, o0, o1, o2):
  ea = ea_ref[...]
  temb = jnp.cos(ea[:, 0:1] * tw_ref[...] + tb_ref[...])
  r = jnp.dot(temb, a2_ref[...], preferred_element_type=jnp.float32, precision=lax.Precision.HIGHEST)
  r += jnp.dot(ea, a1_ref[...], preferred_element_type=jnp.float32, precision=lax.Precision.HIGHEST)
  r += bt_ref[...]
  o0[...] = r[:, 0:16]
  o1[...] = r[:, 16:32]
  o2[...] = r[:, 32:48]


def _eattn(edge_attr, te_w, te_b, a1, a2, bt):
  full = lambda m: (0, 0)
  row = lambda m: (m, 0)
  return pl.pallas_call(
      _eattn_body,
      grid=(_EP // 2048,),
      in_specs=[
          pl.BlockSpec((2048, 16), row),
          pl.BlockSpec((1, _HID), full),
          pl.BlockSpec((1, _HID), full),
          pl.BlockSpec((16, 48), full),
          pl.BlockSpec((_HID, 48), full),
          pl.BlockSpec((1, 48), full),
      ],
      out_specs=[pl.BlockSpec((2048, 16), row)] * 3,
      out_shape=[jax.ShapeDtypeStruct((_EP, 16), jnp.float32)] * 3,
  )(edge_attr, te_w.reshape(1, _HID), te_b.reshape(1, _HID), a1, a2, bt)


def _sc_body(H, C, wd,
             h_hbm, sk_hbm, sb_hbm, ea_hbm, src_hbm, dl_hbm,
             est_hbm, nch_hbm, out_hbm,
             sdst_b, sb_b, ea_b, exal_b, den_b, acc_b, h_b,
             src_i, dl_i, dlv_b, meta_b, sem1, sem2, sem3, sem4, sem5):
  cid = lax.axis_index("c")
  sid = lax.axis_index("s")
  wid = cid * 16 + sid
  nsl = wd // 16
  zv = jnp.zeros((16,), jnp.float32)
  iota = lax.iota(jnp.int32, 16)

  def edge_chunk_logits(g):
    rowi = iota + g * 16
    dl = dl_i[pl.ds(g * 16, 16)]
    isdum = dl >= jnp.int32(_NPB)
    dlv = jnp.where(isdum, jnp.int32(0), dl)
    exs = []
    for j in range(H):
      jf = jnp.full((16,), j, jnp.int32)
      s1 = plsc.load_gather(sb_b, [rowi, jf])
      e1 = plsc.load_gather(ea_b, [rowi, jf])
      s2 = plsc.load_gather(sdst_b, [dlv, jf + 8])
      al = s1 + s2 + e1
      al = jnp.where(al < 0.0, al * jnp.float32(0.2), al)
      exs.append(jnp.where(isdum, jnp.float32(0.0), jnp.exp(al)))
    return rowi, dlv, exs

  def load_chunk(eo, with_h):
    eo = pl.multiple_of(eo, _CH)
    d1 = pltpu.async_copy(src_hbm.at[pl.ds(eo, _CH)], src_i, sem1)
    d2 = pltpu.async_copy(dl_hbm.at[pl.ds(eo, _CH)], dl_i, sem2)
    d3 = pltpu.async_copy(ea_hbm.at[pl.ds(eo, _CH)], ea_b, sem3)
    d1.wait()
    dh = pltpu.async_copy(h_hbm.at[src_i], h_b, sem5) if with_h else None
    d4 = pltpu.async_copy(sb_hbm.at[src_i], sb_b, sem4)
    d2.wait()
    d3.wait()
    d4.wait()
    return dh

  def range_step(rr, carry):
    r = wid + rr * 32
    pltpu.sync_copy(est_hbm.at[r], meta_b)
    estart = meta_b[...][0]
    pltpu.sync_copy(nch_hbm.at[r], meta_b)
    nchunks = meta_b[...][0]
    nstart = pl.multiple_of(r * _NPB, _NPB)

    def zrow(i, _):
      for k in range(nsl):
        acc_b[i, pl.ds(k * 16, 16)] = zv
      den_b[i, pl.ds(0, 16)] = zv
      return 0
    lax.fori_loop(0, _NPB, zrow, 0)
    pltpu.sync_copy(sb_hbm.at[pl.ds(nstart, _NPB)], sdst_b)

    def pass_a(ci, _):
      eo = estart + ci * _CH
      load_chunk(eo, False)
      for g in range(_CH // 16):
        rowi, dlv, exs = edge_chunk_logits(g)
        dlv_b[pl.ds(g * 16, 16)] = dlv
        for j in range(H):
          jf = jnp.full((16,), j, jnp.int32)
          plsc.store_scatter(exal_b, [rowi, jf], exs[j])
      def arow(e, _):
        ef = jnp.full((16,), e, jnp.int32)
        drep = plsc.load_gather(dlv_b, [ef])
        plsc.addupdate_scatter(den_b, [drep, iota], exal_b[e, pl.ds(0, 16)])
        return 0
      lax.fori_loop(0, _CH, arow, 0)
      return 0
    lax.fori_loop(0, nchunks, pass_a, 0)

    def pass_b(ci, _):
      eo = estart + ci * _CH
      dh = load_chunk(eo, True)
      for g in range(_CH // 16):
        rowi, dlv, exs = edge_chunk_logits(g)
        dlv_b[pl.ds(g * 16, 16)] = dlv
        for j in range(H):
          jf = jnp.full((16,), j, jnp.int32)
          dv = plsc.load_gather(den_b, [dlv, jf])
          alpha = exs[j] / (dv + jnp.float32(1e-16))
          plsc.store_scatter(exal_b, [rowi, jf], alpha)
      dh.wait()
      def wrow(e, _):
        ef = jnp.full((16,), e, jnp.int32)
        drep = plsc.load_gather(dlv_b, [ef])
        if C >= 16:
          for j in range(H):
            av = plsc.load_gather(exal_b, [ef, jnp.full((16,), j, jnp.int32)])
            for k in range(C // 16):
              off = j * C + k * 16
              plsc.addupdate_scatter(
                  acc_b, [drep, iota + off], h_b[e, pl.ds(off, 16)] * av)
        else:
          av = plsc.load_gather(exal_b, [ef, jnp.full((16,), 0, jnp.int32)])
          plsc.addupdate_scatter(acc_b, [drep, iota],
                                 h_b[e, pl.ds(0, 16)] * av)
        return 0
      lax.fori_loop(0, _CH, wrow, 0)
      return 0
    lax.fori_loop(0, nchunks, pass_b, 0)

    pltpu.sync_copy(sk_hbm.at[pl.ds(nstart, _NPB)], h_b.at[pl.ds(0, _NPB)])
    def orow(i, _):
      valid = jnp.where(nstart + i < _N, jnp.float32(1.0), jnp.float32(0.0))
      for k in range(nsl):
        sl = pl.ds(k * 16, 16)
        acc_b[i, sl] = jnp.maximum(acc_b[i, sl] + h_b[i, sl], 0.0) * valid
      return 0
    lax.fori_loop(0, _NPB, orow, 0)
    pltpu.sync_copy(acc_b, out_hbm.at[pl.ds(nstart, _NPB)])
    return carry

  lax.fori_loop(0, _RPAD // 32, range_step, 0)


def _make_sc(H, C, wd):
  mesh = plsc.VectorSubcoreMesh(core_axis_name="c", subcore_axis_name="s")
  body = functools.partial(_sc_body, H, C, wd)
  return pl.kernel(
      body,
      out_type=jax.ShapeDtypeStruct((_NPAD, wd), jnp.float32),
      mesh=mesh,
      compiler_params=pltpu.CompilerParams(needs_layout_passes=False),
      scratch_types=[
          pltpu.VMEM((_NPB, 128), jnp.float32),
          pltpu.VMEM((_CH, 128), jnp.float32),
          pltpu.VMEM((_CH, 16), jnp.float32),
          pltpu.VMEM((_CH, 16), jnp.float32),
          pltpu.VMEM((_NPB, 16), jnp.float32),
          pltpu.VMEM((_NPB, wd), jnp.float32),
          pltpu.VMEM((_CH, wd), jnp.float32),
          pltpu.VMEM((_CH,), jnp.int32),
          pltpu.VMEM((_CH,), jnp.int32),
          pltpu.VMEM((_CH,), jnp.int32),
          pltpu.VMEM((16,), jnp.int32),
          pltpu.SemaphoreType.DMA,
          pltpu.SemaphoreType.DMA,
          pltpu.SemaphoreType.DMA,
          pltpu.SemaphoreType.DMA,
          pltpu.SemaphoreType.DMA,
      ],
  )


_SC_KERNELS = {}


def _sc_call(H, C, wd, *args):
  key = (H, C, wd)
  if key not in _SC_KERNELS:
    _SC_KERNELS[key] = _make_sc(H, C, wd)
  return _SC_KERNELS[key](*args)


def kernel(x, edge_index, edge_attr, te_w, te_b, emlp_W, emlp_b,
           W0, Wedge0, asrc0, adst0, aedge0, b0, sW0, sb0, g0, be0,
           W1, Wedge1, asrc1, adst1, aedge1, b1, sW1, sb1, g1, be1,
           W2, Wedge2, asrc2, adst2, aedge2, b2, sW2, sb2, g2, be2):
  layers = [
      (W0, Wedge0, asrc0, adst0, aedge0, b0, sW0, sb0, g0, be0),
      (W1, Wedge1, asrc1, adst1, aedge1, b1, sW1, sb1, g1, be1),
      (W2, Wedge2, asrc2, adst2, aedge2, b2, sW2, sb2, g2, be2),
  ]

  src = edge_index[0]
  dst = edge_index[1]

  perm = jnp.argsort(dst)
  dst_s = dst[perm]
  src_s = src[perm]
  rid = dst_s // _NPB
  dstloc = dst_s - rid * _NPB
  counts = jnp.zeros((_RPAD,), jnp.int32).at[rid].add(1)
  cntp = ((counts + _CH - 1) // _CH) * _CH
  starts = jnp.concatenate([jnp.zeros((1,), jnp.int32),
                            jnp.cumsum(cntp)[:-1].astype(jnp.int32)])
  fidx = jnp.cumsum(counts).astype(jnp.int32) - counts
  pos = starts[rid] + (jnp.arange(_E, dtype=jnp.int32) - fidx[rid])
  src_pad = jnp.zeros((_EP,), jnp.int32).at[pos].set(src_s)
  dl_pad = jnp.full((_EP,), _NPB, jnp.int32).at[pos].set(dstloc)
  ea_pad = jnp.zeros((_EP, 16), jnp.float32).at[pos].set(edge_attr[perm])
  est_arr = jnp.broadcast_to(starts[:, None], (_RPAD, 16)).astype(jnp.int32)
  nch_arr = jnp.broadcast_to((cntp // _CH)[:, None], (_RPAD, 16)).astype(jnp.int32)

  a1s, a2s, bts = [], [], []
  for i in range(3):
    H, C = _HEADS[i], _OUTC[i]
    We = layers[i][1].reshape(_HID, H, C)
    Mi = jnp.einsum("khc,hc->kh", We, layers[i][4])
    a1 = jnp.zeros((16, 16), jnp.float32).at[1:16, :H].set((emlp_W[:15] @ Mi))
    a2 = jnp.zeros((_HID, 16), jnp.float32).at[:, :H].set(emlp_W[15:] @ Mi)
    bt = jnp.zeros((16,), jnp.float32).at[:H].set(emlp_b @ Mi)
    a1s.append(a1); a2s.append(a2); bts.append(bt)
  a1cat = jnp.concatenate(a1s, axis=1)
  a2cat = jnp.concatenate(a2s, axis=1)
  btcat = jnp.concatenate(bts).reshape(1, 48)
  ea0, ea1, ea2 = _eattn(ea_pad, te_w, te_b, a1cat, a2cat, btcat)
  eattns = [ea0, ea1, ea2]

  xp = jnp.pad(x, ((0, _NPAD - _N), (0, 0)))
  for i in range(3):
    W, _, asrc, adst, _, b, sW, sb, g, be = layers[i]
    H, C, ind = _HEADS[i], _OUTC[i], _IND[i]
    wd = max(H * C, 128)

    mu, var = _stats(xp)
    s = g * lax.rsqrt(var + 1e-5)
    shift = be - mu * s
    wh = s[:, None] * W
    bh = shift @ W
    ws = s[:, None] * sW
    bs = shift @ sW + sb + b
    if H * C < wd:
      wh = jnp.pad(wh, ((0, 0), (0, wd - H * C)))
      bh = jnp.pad(bh, (0, wd - H * C))
      ws = jnp.pad(ws, ((0, 0), (0, wd - H * C)))
      bs = jnp.pad(bs, (0, wd - H * C))
    masrc = jnp.einsum("khc,hc->kh", wh[:, :H * C].reshape(ind, H, C), asrc)
    madst = jnp.einsum("khc,hc->kh", wh[:, :H * C].reshape(ind, H, C), adst)
    bsrc = jnp.einsum("hc,hc->h", bh[:H * C].reshape(H, C), asrc)
    bdst = jnp.einsum("hc,hc->h", bh[:H * C].reshape(H, C), adst)
    wm = jnp.zeros((ind, 128), jnp.float32).at[:, :H].set(masrc).at[:, 8:8 + H].set(madst)
    bm = jnp.zeros((128,), jnp.float32).at[:H].set(bsrc).at[8:8 + H].set(bdst)

    h, skipb, sboth = _mm(xp, wh, bh.reshape(1, wd), ws, bs.reshape(1, wd),
                          wm, bm.reshape(1, 128))
    xp = _sc_call(H, C, wd, h, skipb, sboth, eattns[i],
                  src_pad, dl_pad, est_arr, nch_arr)

  return xp[:_N, :2]
